# baseline (speedup 1.0000x reference)
import functools
import jax
import jax.numpy as jnp
from jax import lax
from jax.experimental import pallas as pl
from jax.experimental.pallas import tpu as pltpu
from jax.experimental.pallas import tpu_sc as plsc

S = 2048
B = 4
D = 1024
NC = 2
NS = 16
NW = NC * NS
ROWS_W = S // NW
CHS = 8
NCH = ROWS_W // CHS
XR = CHS * B
NV = D // 16


def _sc_body(x_hbm, e_hbm, o_hbm, xbuf, ebuf, xsem, esem, osem):
    wid = lax.axis_index("s") * NC + lax.axis_index("c")
    s0 = wid * ROWS_W

    def in_x(c, slot):
        return pltpu.make_async_copy(
            x_hbm.at[pl.ds((s0 + c * CHS) * B, XR)], xbuf.at[slot],
            xsem.at[slot])

    def in_e(c, slot):
        return pltpu.make_async_copy(
            e_hbm.at[pl.ds((s0 + c * CHS + 1) * 8, CHS * 8)], ebuf.at[slot],
            esem.at[slot])

    def out_o(c, slot):
        return pltpu.make_async_copy(
            xbuf.at[slot], o_hbm.at[pl.ds((s0 + c * CHS) * B, XR)],
            osem.at[slot])

    in_x(0, 0).start()
    in_e(0, 0).start()
    for c in range(NCH):
        slot = c & 1
        if c + 1 < NCH:
            in_x(c + 1, 1 - slot).start()
            in_e(c + 1, 1 - slot).start()
        in_x(c, slot).wait()
        in_e(c, slot).wait()
        if c >= 2:
            out_o(c - 2, slot).wait()

        def dv_body(dv, _, slot=slot):
            dv8 = dv // 8
            dl16 = (dv % 8) * 16
            for sl in range(CHS):
                e = ebuf[slot, sl * 8 + dv8, pl.ds(dl16, 16)]
                for b in range(B):
                    r = sl * B + b
                    xbuf[slot, r, pl.ds(dv * 16, 16)] = (
                        xbuf[slot, r, pl.ds(dv * 16, 16)] + e)
            return 0

        lax.fori_loop(0, NV, dv_body, 0)
        out_o(c, slot).start()
    out_o(NCH - 2, 0).wait()
    out_o(NCH - 1, 1).wait()


_sc_kernel = functools.partial(
    pl.kernel,
    out_type=jax.ShapeDtypeStruct((S * B, D), jnp.float32),
    mesh=plsc.VectorSubcoreMesh(core_axis_name="c", subcore_axis_name="s"),
    scratch_types=[
        pltpu.VMEM((2, XR, D), jnp.float32),
        pltpu.VMEM((2, CHS * 8, D // 8), jnp.float32),
        pltpu.SemaphoreType.DMA((2,)),
        pltpu.SemaphoreType.DMA((2,)),
        pltpu.SemaphoreType.DMA((2,)),
    ],
)(_sc_body)


def kernel(inputs, pos_emb):
    S_, B_, D_ = inputs.shape
    x2 = inputs.reshape(S_ * B_, D_)
    e2 = pos_emb.reshape(pos_emb.shape[0] * 8, D_ // 8)
    out = _sc_kernel(x2, e2)
    return out.reshape(S_, B_, D_)


# speedup vs baseline: 2.4531x; 2.4531x over previous
import functools
import jax
import jax.numpy as jnp
from jax import lax
from jax.experimental import pallas as pl
from jax.experimental.pallas import tpu as pltpu
from jax.experimental.pallas import tpu_sc as plsc

S = 2048
B = 4
D = 1024
NC = 2
NS = 16
NW = NC * NS
ROWS_W = S // NW
CHS = 8
NCH = ROWS_W // CHS
NV = D // 16


def _sc_body(x_hbm, e_hbm, o_hbm, xbuf, eabuf, ebbuf, xsem, easem, ebsem,
             osem):
    wid = lax.axis_index("s") * NC + lax.axis_index("c")
    s0 = wid * ROWS_W

    def in_x(c, slot):
        return pltpu.make_async_copy(
            x_hbm.at[pl.ds(s0 + c * CHS, CHS)], xbuf.at[slot], xsem.at[slot])

    def in_ea(c, slot):
        return pltpu.make_async_copy(
            e_hbm.at[pl.ds(s0 + c * CHS, CHS)], eabuf.at[slot],
            easem.at[slot])

    def in_eb(c, slot):
        return pltpu.make_async_copy(
            e_hbm.at[pl.ds(s0 + c * CHS + CHS, 1)], ebbuf.at[slot],
            ebsem.at[slot])

    def out_o(c, slot):
        return pltpu.make_async_copy(
            xbuf.at[slot], o_hbm.at[pl.ds(s0 + c * CHS, CHS)], osem.at[slot])

    def start_in(c, slot):
        in_x(c, slot).start()
        in_ea(c, slot).start()
        in_eb(c, slot).start()

    start_in(0, 0)
    for c in range(NCH):
        slot = c & 1
        if c + 1 < NCH:
            start_in(c + 1, 1 - slot)
        in_x(c, slot).wait()
        in_ea(c, slot).wait()
        in_eb(c, slot).wait()
        if c >= 2:
            out_o(c - 2, slot).wait()

        def dv_body(dv, _, slot=slot):
            dd = pl.ds(dv * 16, 16)
            for sl in range(CHS):
                if sl < CHS - 1:
                    e = eabuf[slot, sl + 1, dd]
                else:
                    e = ebbuf[slot, 0, dd]
                for b in range(B):
                    xbuf[slot, sl, b, dd] = xbuf[slot, sl, b, dd] + e
            return 0

        lax.fori_loop(0, NV, dv_body, 0)
        out_o(c, slot).start()
    out_o(NCH - 2, 0).wait()
    out_o(NCH - 1, 1).wait()


_sc_kernel = functools.partial(
    pl.kernel,
    out_type=jax.ShapeDtypeStruct((S, B, D), jnp.float32),
    mesh=plsc.VectorSubcoreMesh(core_axis_name="c", subcore_axis_name="s"),
    scratch_types=[
        pltpu.VMEM((2, CHS, B, D), jnp.float32),
        pltpu.VMEM((2, CHS, D), jnp.float32),
        pltpu.VMEM((2, 1, D), jnp.float32),
        pltpu.SemaphoreType.DMA((2,)),
        pltpu.SemaphoreType.DMA((2,)),
        pltpu.SemaphoreType.DMA((2,)),
        pltpu.SemaphoreType.DMA((2,)),
    ],
)(_sc_body)


def kernel(inputs, pos_emb):
    return _sc_kernel(inputs, pos_emb)
